# baseline (speedup 1.0000x reference)
import functools

import jax
import jax.numpy as jnp
from jax import lax
from jax.experimental import pallas as pl
from jax.experimental.pallas import tpu as pltpu
from jax.experimental.pallas import tpu_sc as plsc

NN = 10000
NPK = 1250
NP = 10112
NPP = NP // 8
EE = 320000
CH = 128
NROW = EE // CH
NW = 32
BASE = 78
GR = 13
NGR = BASE // GR
GB = GR * CH
RS = NP // 16
OPc = 304
OO = 300


def _seg_body(vals, edges, zeros_h, out_acc, src_v, dst_v, rows_v, acc,
              sem_g, sem_sv):
    cid = lax.axis_index("c")
    sid = lax.axis_index("s")
    wid = sid * 2 + cid
    pltpu.sync_copy(zeros_h.at[pl.ds(sid * RS, RS)], acc.at[pl.ds(sid * RS, RS)])
    pltpu.sync_copy(edges.at[0, pl.ds(wid * BASE * CH, BASE * CH)],
                    src_v.at[pl.ds(0, BASE * CH)])
    pltpu.sync_copy(edges.at[1, pl.ds(wid * BASE * CH, BASE * CH)],
                    dst_v.at[pl.ds(0, BASE * CH)])

    @pl.when(wid < NROW - NW * BASE)
    def _():
        pltpu.sync_copy(edges.at[0, pl.ds(NW * BASE * CH + wid * CH, CH)],
                        src_v.at[pl.ds(BASE * CH, CH)])
        pltpu.sync_copy(edges.at[1, pl.ds(NW * BASE * CH + wid * CH, CH)],
                        dst_v.at[pl.ds(BASE * CH, CH)])

    plsc.subcore_barrier()

    def gather(g, buf):
        pltpu.async_copy(vals.at[src_v.at[pl.ds(g * GB, GB)]],
                         rows_v.at[pl.ds(buf * GB, GB)], sem_g)

    def drain_gather():
        pltpu.make_async_copy(
            vals.at[src_v.at[pl.ds(0, GB)]], rows_v.at[pl.ds(0, GB)], sem_g).wait()

    def scatter(g, buf):
        pltpu.async_copy(rows_v.at[pl.ds(buf * GB, GB)],
                         acc.at[dst_v.at[pl.ds(g * GB, GB)]], sem_sv, add=True)

    def drain_scatter():
        pltpu.make_async_copy(
            rows_v.at[pl.ds(0, GB)], acc.at[dst_v.at[pl.ds(0, GB)]], sem_sv).wait()

    gather(0, 0)
    for g in range(NGR):
        buf = g % 2
        drain_gather()
        if g + 1 < NGR:
            if g >= 1:
                drain_scatter()
            gather(g + 1, 1 - buf)
        scatter(g, buf)
    drain_scatter()
    drain_scatter()

    @pl.when(wid < NROW - NW * BASE)
    def _():
        pltpu.async_copy(vals.at[src_v.at[pl.ds(BASE * CH, CH)]],
                         rows_v.at[pl.ds(0, CH)], sem_g).wait()
        pltpu.sync_copy(rows_v.at[pl.ds(0, CH)],
                        acc.at[dst_v.at[pl.ds(BASE * CH, CH)]], add=True)

    plsc.subcore_barrier()
    pltpu.sync_copy(acc.at[pl.ds(sid * RS, RS)], out_acc.at[cid, pl.ds(sid * RS, RS)])


@functools.cache
def _make_seg():
    mesh = plsc.VectorSubcoreMesh(
        core_axis_name="c", subcore_axis_name="s", num_cores=2, num_subcores=16
    )
    return pl.kernel(
        _seg_body,
        out_type=jax.ShapeDtypeStruct((2, NP, 16), jnp.float32),
        mesh=mesh,
        scratch_types=[
            pltpu.VMEM(((BASE + 1) * CH,), jnp.int32),
            pltpu.VMEM(((BASE + 1) * CH,), jnp.int32),
            pltpu.VMEM((2 * GB, 16), jnp.float32),
            pltpu.VMEM_SHARED((NP, 16), jnp.float32),
            pltpu.SemaphoreType.DMA,
            pltpu.SemaphoreType.DMA,
        ],
        compiler_params=pltpu.CompilerParams(use_tc_tiling_on_sc=False),
    )


def _permsc_body(edges3, zeros_h, ones_h, out, out_deg, buf, ones_v, accd, sem_d):
    cid = lax.axis_index("c")
    sid = lax.axis_index("s")
    wid = sid * 2 + cid
    extra = wid < NROW - NW * BASE
    nrow = BASE + 1

    pltpu.sync_copy(zeros_h.at[pl.ds(sid * RS, RS)], accd.at[pl.ds(sid * RS, RS)])
    pltpu.sync_copy(ones_h, ones_v)

    def row(r, carry):
        for k in range(8):
            v = buf[r, pl.ds(16 * k, 16)]
            q = (v.astype(jnp.float32) * (1.0 / NPK)).astype(jnp.int32)
            buf[r, pl.ds(16 * k, 16)] = (v - q * NPK) * 8 + q
        return carry

    for j in range(2):
        pltpu.sync_copy(edges3.at[pl.ds(wid * BASE, BASE), j], buf.at[pl.ds(0, BASE)])

        @pl.when(extra)
        def _():
            pltpu.sync_copy(edges3.at[pl.ds(NW * BASE + wid, 1), j],
                            buf.at[pl.ds(BASE, 1)])

        if j == 0:
            plsc.subcore_barrier()

        lax.fori_loop(0, BASE, row, 0)

        @pl.when(extra)
        def _():
            lax.fori_loop(BASE, BASE + 1, row, 0)

        pltpu.sync_copy(buf.at[pl.ds(0, BASE)], out.at[j, pl.ds(wid * BASE, BASE)])

        @pl.when(extra)
        def _():
            pltpu.sync_copy(buf.at[pl.ds(BASE, 1)],
                            out.at[j, pl.ds(NW * BASE + wid, 1)])

        if j == 1:
            def fire(r, carry):
                pltpu.async_copy(ones_v, accd.at[buf.at[r]], sem_d, add=True)
                return carry

            def drain(r, carry):
                pltpu.make_async_copy(ones_v, accd.at[buf.at[0]], sem_d).wait()
                return carry

            lax.fori_loop(0, BASE, fire, 0)

            @pl.when(extra)
            def _():
                lax.fori_loop(BASE, nrow, fire, 0)

            lax.fori_loop(0, BASE, drain, 0)

            @pl.when(extra)
            def _():
                lax.fori_loop(BASE, nrow, drain, 0)

    plsc.subcore_barrier()
    pltpu.sync_copy(accd.at[pl.ds(sid * RS, RS)], out_deg.at[cid, pl.ds(sid * RS, RS)])


@functools.cache
def _make_perm():
    mesh = plsc.VectorSubcoreMesh(
        core_axis_name="c", subcore_axis_name="s", num_cores=2, num_subcores=16
    )
    return pl.kernel(
        _permsc_body,
        out_type=(
            jax.ShapeDtypeStruct((2, NROW, CH), jnp.int32),
            jax.ShapeDtypeStruct((2, NP, 16), jnp.float32),
        ),
        mesh=mesh,
        scratch_types=[
            pltpu.VMEM((BASE + 1, CH), jnp.int32),
            pltpu.VMEM((CH, 16), jnp.float32),
            pltpu.VMEM_SHARED((NP, 16), jnp.float32),
            pltpu.SemaphoreType.DMA,
        ],
        compiler_params=pltpu.CompilerParams(use_tc_tiling_on_sc=False),
    )


def _proj_body(x_ref, wl_ref, wr_ref, xl_ref, xr_ref):
    xv = x_ref[...]
    dn = (((1,), (1,)), ((), ()))
    xls, xrs = [], []
    for a in range(8):
        xa = xv[a]
        xls.append(lax.dot_general(xa, wl_ref[...], dn, preferred_element_type=jnp.float32))
        xrs.append(lax.dot_general(xa, wr_ref[...], dn, preferred_element_type=jnp.float32))
    xl_ref[...] = jnp.concatenate(xls, axis=1)
    xr_ref[...] = jnp.concatenate(xrs, axis=1)


_proj = pl.pallas_call(
    _proj_body,
    out_shape=(
        jax.ShapeDtypeStruct((NPK, 128), jnp.float32),
        jax.ShapeDtypeStruct((NPK, 128), jnp.float32),
    ),
)


def _h_body(a_ref, d_ref, xr_ref, b1_ref, hp_ref, dinv_ref):
    deg = d_ref[0, :NPK] + d_ref[1, :NPK]
    dinv = 1.0 / jnp.maximum(deg, 1.0)
    agg = a_ref[0, :NPK] + a_ref[1, :NPK]
    hp_ref[...] = jnp.maximum(agg * dinv + b1_ref[...] + xr_ref[...], 0.0)
    dinv_ref[...] = dinv


_hcomb = pl.pallas_call(
    _h_body,
    out_shape=(
        jax.ShapeDtypeStruct((NPK, 128), jnp.float32),
        jax.ShapeDtypeStruct((NPK, 128), jnp.float32),
    ),
)


def _out_body(a_ref, dinv_ref, h_ref, w2l_ref, w2r_ref, b2_ref, o_ref):
    m2p = (a_ref[0, :NPK] + a_ref[1, :NPK]) * dinv_ref[...]
    hpv = h_ref[...]
    dn = (((1,), (1,)), ((), ()))
    cols = []
    for a in range(8):
        m2a = m2p[:, 16 * a:16 * (a + 1)]
        ha = hpv[:, 16 * a:16 * (a + 1)]
        z = (lax.dot_general(w2l_ref[...], m2a, dn, preferred_element_type=jnp.float32)
             + lax.dot_general(w2r_ref[...], ha, dn, preferred_element_type=jnp.float32)
             + b2_ref[...])
        m = jnp.max(z, axis=0, keepdims=True)
        lse = jnp.log(jnp.sum(jnp.exp(z - m), axis=0, keepdims=True)) + m
        cols.append(z - lse)
    o_ref[...] = jnp.concatenate(cols, axis=1)


_outk = pl.pallas_call(
    _out_body,
    out_shape=jax.ShapeDtypeStruct((OPc, NN), jnp.float32),
)


def kernel(x, edge_index, W1l, b1, W1r, W2l, b2, W2r):
    edges3 = edge_index.astype(jnp.int32).reshape(2, NROW, CH).swapaxes(0, 1)
    zeros_h = jnp.zeros((NP, 16), jnp.float32)
    ones_h = jnp.ones((CH, 16), jnp.float32)
    pedges, degp = _make_perm()(edges3, zeros_h, ones_h)
    edges = pedges.reshape(2, EE)

    xlp, xrp = _proj(x.reshape(8, NPK, 128), W1l, W1r)
    agg1p = _make_seg()(xlp.reshape(NN, 16), edges, zeros_h)
    hp, dinvp = _hcomb(
        agg1p.reshape(2, NPP, 128), degp.reshape(2, NPP, 128), xrp,
        jnp.tile(b1, 8).reshape(1, 128))
    agg2p = _make_seg()(hp.reshape(NN, 16), edges, zeros_h)

    w2l_p = jnp.zeros((OPc, 16), jnp.float32).at[:OO].set(W2l)
    w2r_p = jnp.zeros((OPc, 16), jnp.float32).at[:OO].set(W2r)
    b2c = jnp.full((OPc, 1), -1e30, jnp.float32).at[:OO, 0].set(b2)
    outT = _outk(agg2p.reshape(2, NPP, 128), dinvp, hp, w2l_p, w2r_p, b2c)
    return outT.T[:, :OO]


# speedup vs baseline: 1.0354x; 1.0354x over previous
import functools

import jax
import jax.numpy as jnp
from jax import lax
from jax.experimental import pallas as pl
from jax.experimental.pallas import tpu as pltpu
from jax.experimental.pallas import tpu_sc as plsc

NN = 10000
NPK = 1250
NP = 10112
NPP = NP // 8
EE = 320000
CH = 128
NROW = EE // CH
NW = 32
BASE = 78
GR = 13
NGR = BASE // GR
GB = GR * CH
RS = NP // 16
OPc = 304
OO = 300


def _seg_body(with_deg, vals, edges, zeros_h, ones_h, *rest):
    if with_deg:
        (out_acc, out_deg, src_v, dst_v, rows_v, ones_v, acc, accd,
         sem_g, sem_sv, sem_sd) = rest
    else:
        out_acc, src_v, dst_v, rows_v, acc, sem_g, sem_sv = rest
    cid = lax.axis_index("c")
    sid = lax.axis_index("s")
    wid = sid * 2 + cid
    pltpu.sync_copy(zeros_h.at[pl.ds(sid * RS, RS)], acc.at[pl.ds(sid * RS, RS)])
    if with_deg:
        pltpu.sync_copy(zeros_h.at[pl.ds(sid * RS, RS)], accd.at[pl.ds(sid * RS, RS)])
        pltpu.sync_copy(ones_h, ones_v)
    pltpu.sync_copy(edges.at[0, pl.ds(wid * BASE * CH, BASE * CH)],
                    src_v.at[pl.ds(0, BASE * CH)])
    pltpu.sync_copy(edges.at[1, pl.ds(wid * BASE * CH, BASE * CH)],
                    dst_v.at[pl.ds(0, BASE * CH)])

    @pl.when(wid < NROW - NW * BASE)
    def _():
        pltpu.sync_copy(edges.at[0, pl.ds(NW * BASE * CH + wid * CH, CH)],
                        src_v.at[pl.ds(BASE * CH, CH)])
        pltpu.sync_copy(edges.at[1, pl.ds(NW * BASE * CH + wid * CH, CH)],
                        dst_v.at[pl.ds(BASE * CH, CH)])

    plsc.subcore_barrier()

    def gather(g, buf):
        pltpu.async_copy(vals.at[src_v.at[pl.ds(g * GB, GB)]],
                         rows_v.at[pl.ds(buf * GB, GB)], sem_g)

    def drain_gather():
        pltpu.make_async_copy(
            vals.at[src_v.at[pl.ds(0, GB)]], rows_v.at[pl.ds(0, GB)], sem_g).wait()

    def scatter(g, buf):
        pltpu.async_copy(rows_v.at[pl.ds(buf * GB, GB)],
                         acc.at[dst_v.at[pl.ds(g * GB, GB)]], sem_sv, add=True)
        if with_deg:
            pltpu.async_copy(ones_v, accd.at[dst_v.at[pl.ds(g * GB, GB)]],
                             sem_sd, add=True)

    def drain_scatter():
        pltpu.make_async_copy(
            rows_v.at[pl.ds(0, GB)], acc.at[dst_v.at[pl.ds(0, GB)]], sem_sv).wait()
        if with_deg:
            pltpu.make_async_copy(
                ones_v, accd.at[dst_v.at[pl.ds(0, GB)]], sem_sd).wait()

    gather(0, 0)
    for g in range(NGR):
        buf = g % 2
        drain_gather()
        if g + 1 < NGR:
            if g >= 1:
                drain_scatter()
            gather(g + 1, 1 - buf)
        scatter(g, buf)
    drain_scatter()
    drain_scatter()

    @pl.when(wid < NROW - NW * BASE)
    def _():
        pltpu.async_copy(vals.at[src_v.at[pl.ds(BASE * CH, CH)]],
                         rows_v.at[pl.ds(0, CH)], sem_g).wait()
        pltpu.sync_copy(rows_v.at[pl.ds(0, CH)],
                        acc.at[dst_v.at[pl.ds(BASE * CH, CH)]], add=True)
        if with_deg:
            pltpu.sync_copy(ones_v.at[pl.ds(0, CH)],
                            accd.at[dst_v.at[pl.ds(BASE * CH, CH)]], add=True)

    plsc.subcore_barrier()
    pltpu.sync_copy(acc.at[pl.ds(sid * RS, RS)], out_acc.at[cid, pl.ds(sid * RS, RS)])
    if with_deg:
        pltpu.sync_copy(accd.at[pl.ds(sid * RS, RS)],
                        out_deg.at[cid, pl.ds(sid * RS, RS)])


@functools.cache
def _make_seg(with_deg):
    mesh = plsc.VectorSubcoreMesh(
        core_axis_name="c", subcore_axis_name="s", num_cores=2, num_subcores=16
    )
    outs = [jax.ShapeDtypeStruct((2, NP, 16), jnp.float32)]
    scratch = [
        pltpu.VMEM(((BASE + 1) * CH,), jnp.int32),
        pltpu.VMEM(((BASE + 1) * CH,), jnp.int32),
        pltpu.VMEM((2 * GB, 16), jnp.float32),
    ]
    if with_deg:
        outs.append(jax.ShapeDtypeStruct((2, NP, 16), jnp.float32))
        scratch.append(pltpu.VMEM((GB, 16), jnp.float32))
    scratch.append(pltpu.VMEM_SHARED((NP, 16), jnp.float32))
    if with_deg:
        scratch.append(pltpu.VMEM_SHARED((NP, 16), jnp.float32))
    scratch.append(pltpu.SemaphoreType.DMA)
    scratch.append(pltpu.SemaphoreType.DMA)
    if with_deg:
        scratch.append(pltpu.SemaphoreType.DMA)
    return pl.kernel(
        functools.partial(_seg_body, with_deg),
        out_type=tuple(outs) if with_deg else outs[0],
        mesh=mesh,
        scratch_types=scratch,
        compiler_params=pltpu.CompilerParams(use_tc_tiling_on_sc=False),
    )


def _permsc_body(edges3, out, buf):
    cid = lax.axis_index("c")
    sid = lax.axis_index("s")
    wid = sid * 2 + cid
    extra = wid < NROW - NW * BASE

    def row(r, carry):
        for k in range(8):
            v = buf[r, pl.ds(16 * k, 16)]
            q = (v.astype(jnp.float32) * (1.0 / NPK)).astype(jnp.int32)
            buf[r, pl.ds(16 * k, 16)] = (v - q * NPK) * 8 + q
        return carry

    for j in range(2):
        pltpu.sync_copy(edges3.at[pl.ds(wid * BASE, BASE), j], buf.at[pl.ds(0, BASE)])

        @pl.when(extra)
        def _():
            pltpu.sync_copy(edges3.at[pl.ds(NW * BASE + wid, 1), j],
                            buf.at[pl.ds(BASE, 1)])

        lax.fori_loop(0, BASE, row, 0)

        @pl.when(extra)
        def _():
            lax.fori_loop(BASE, BASE + 1, row, 0)

        pltpu.sync_copy(buf.at[pl.ds(0, BASE)], out.at[j, pl.ds(wid * BASE, BASE)])

        @pl.when(extra)
        def _():
            pltpu.sync_copy(buf.at[pl.ds(BASE, 1)],
                            out.at[j, pl.ds(NW * BASE + wid, 1)])


@functools.cache
def _make_perm():
    mesh = plsc.VectorSubcoreMesh(
        core_axis_name="c", subcore_axis_name="s", num_cores=2, num_subcores=16
    )
    return pl.kernel(
        _permsc_body,
        out_type=jax.ShapeDtypeStruct((2, NROW, CH), jnp.int32),
        mesh=mesh,
        scratch_types=[pltpu.VMEM((BASE + 1, CH), jnp.int32)],
        compiler_params=pltpu.CompilerParams(use_tc_tiling_on_sc=False),
    )


def _proj_body(x_ref, wl_ref, wr_ref, xl_ref, xr_ref):
    xv = x_ref[...]
    dn = (((1,), (1,)), ((), ()))
    xls, xrs = [], []
    for a in range(8):
        xa = xv[a]
        xls.append(lax.dot_general(xa, wl_ref[...], dn, preferred_element_type=jnp.float32))
        xrs.append(lax.dot_general(xa, wr_ref[...], dn, preferred_element_type=jnp.float32))
    xl_ref[...] = jnp.concatenate(xls, axis=1)
    xr_ref[...] = jnp.concatenate(xrs, axis=1)


_proj = pl.pallas_call(
    _proj_body,
    out_shape=(
        jax.ShapeDtypeStruct((NPK, 128), jnp.float32),
        jax.ShapeDtypeStruct((NPK, 128), jnp.float32),
    ),
)


def _h_body(a_ref, d_ref, xr_ref, b1_ref, hp_ref, dinv_ref):
    deg = d_ref[0, :NPK] + d_ref[1, :NPK]
    dinv = 1.0 / jnp.maximum(deg, 1.0)
    agg = a_ref[0, :NPK] + a_ref[1, :NPK]
    hp_ref[...] = jnp.maximum(agg * dinv + b1_ref[...] + xr_ref[...], 0.0)
    dinv_ref[...] = dinv


_hcomb = pl.pallas_call(
    _h_body,
    out_shape=(
        jax.ShapeDtypeStruct((NPK, 128), jnp.float32),
        jax.ShapeDtypeStruct((NPK, 128), jnp.float32),
    ),
)


def _out_body(a_ref, dinv_ref, h_ref, w2l_ref, w2r_ref, b2_ref, o_ref):
    m2p = (a_ref[0, :NPK] + a_ref[1, :NPK]) * dinv_ref[...]
    hpv = h_ref[...]
    dn = (((1,), (1,)), ((), ()))
    cols = []
    for a in range(8):
        m2a = m2p[:, 16 * a:16 * (a + 1)]
        ha = hpv[:, 16 * a:16 * (a + 1)]
        z = (lax.dot_general(w2l_ref[...], m2a, dn, preferred_element_type=jnp.float32)
             + lax.dot_general(w2r_ref[...], ha, dn, preferred_element_type=jnp.float32)
             + b2_ref[...])
        m = jnp.max(z, axis=0, keepdims=True)
        lse = jnp.log(jnp.sum(jnp.exp(z - m), axis=0, keepdims=True)) + m
        cols.append(z - lse)
    o_ref[...] = jnp.concatenate(cols, axis=1)


_outk = pl.pallas_call(
    _out_body,
    out_shape=jax.ShapeDtypeStruct((OPc, NN), jnp.float32),
)


def kernel(x, edge_index, W1l, b1, W1r, W2l, b2, W2r):
    edges3 = edge_index.astype(jnp.int32).reshape(2, NROW, CH).swapaxes(0, 1)
    zeros_h = jnp.zeros((NP, 16), jnp.float32)
    ones_h = jnp.ones((GB, 16), jnp.float32)
    edges = _make_perm()(edges3).reshape(2, EE)

    xlp, xrp = _proj(x.reshape(8, NPK, 128), W1l, W1r)
    agg1p, degp = _make_seg(True)(xlp.reshape(NN, 16), edges, zeros_h, ones_h)
    hp, dinvp = _hcomb(
        agg1p.reshape(2, NPP, 128), degp.reshape(2, NPP, 128), xrp,
        jnp.tile(b1, 8).reshape(1, 128))
    agg2p = _make_seg(False)(hp.reshape(NN, 16), edges, zeros_h, ones_h)

    w2l_p = jnp.zeros((OPc, 16), jnp.float32).at[:OO].set(W2l)
    w2r_p = jnp.zeros((OPc, 16), jnp.float32).at[:OO].set(W2r)
    b2c = jnp.full((OPc, 1), -1e30, jnp.float32).at[:OO, 0].set(b2)
    outT = _outk(agg2p.reshape(2, NPP, 128), dinvp, hp, w2l_p, w2r_p, b2c)
    return outT.T[:, :OO]


# speedup vs baseline: 1.0681x; 1.0315x over previous
import functools

import jax
import jax.numpy as jnp
from jax import lax
from jax.experimental import pallas as pl
from jax.experimental.pallas import tpu as pltpu
from jax.experimental.pallas import tpu_sc as plsc

NN = 10000
NPK = 1250
NP = 10112
NPP = NP // 8
EE = 320000
CH = 128
NROW = EE // CH
NW = 32
BASE = 78
GR = 13
NGR = BASE // GR
GB = GR * CH
RS = NP // 16
OPc = 304
OO = 300


def _seg_body(with_deg, vals, edges, zeros_h, ones_h, *rest):
    if with_deg:
        (out_acc, out_deg, src_v, dst_v, rows_v, ones_v, acc, accd,
         sem_g, sem_sv, sem_sd) = rest
    else:
        out_acc, src_v, dst_v, rows_v, acc, vals_s, sem_g, sem_sv = rest
    cid = lax.axis_index("c")
    sid = lax.axis_index("s")
    wid = sid * 2 + cid
    pltpu.sync_copy(zeros_h.at[pl.ds(sid * RS, RS)], acc.at[pl.ds(sid * RS, RS)])
    if with_deg:
        pltpu.sync_copy(zeros_h.at[pl.ds(sid * RS, RS)], accd.at[pl.ds(sid * RS, RS)])
        pltpu.sync_copy(ones_h, ones_v)
    else:
        pltpu.sync_copy(vals.at[pl.ds(sid * (NN // 16), NN // 16)],
                        vals_s.at[pl.ds(sid * (NN // 16), NN // 16)])
    pltpu.sync_copy(edges.at[0, pl.ds(wid * BASE * CH, BASE * CH)],
                    src_v.at[pl.ds(0, BASE * CH)])
    pltpu.sync_copy(edges.at[1, pl.ds(wid * BASE * CH, BASE * CH)],
                    dst_v.at[pl.ds(0, BASE * CH)])

    @pl.when(wid < NROW - NW * BASE)
    def _():
        pltpu.sync_copy(edges.at[0, pl.ds(NW * BASE * CH + wid * CH, CH)],
                        src_v.at[pl.ds(BASE * CH, CH)])
        pltpu.sync_copy(edges.at[1, pl.ds(NW * BASE * CH + wid * CH, CH)],
                        dst_v.at[pl.ds(BASE * CH, CH)])

    plsc.subcore_barrier()

    table = vals if with_deg else vals_s

    def gather(g, buf):
        pltpu.async_copy(table.at[src_v.at[pl.ds(g * GB, GB)]],
                         rows_v.at[pl.ds(buf * GB, GB)], sem_g)

    def drain_gather():
        pltpu.make_async_copy(
            table.at[src_v.at[pl.ds(0, GB)]], rows_v.at[pl.ds(0, GB)], sem_g).wait()

    def scatter(g, buf):
        pltpu.async_copy(rows_v.at[pl.ds(buf * GB, GB)],
                         acc.at[dst_v.at[pl.ds(g * GB, GB)]], sem_sv, add=True)
        if with_deg:
            pltpu.async_copy(ones_v, accd.at[dst_v.at[pl.ds(g * GB, GB)]],
                             sem_sd, add=True)

    def drain_scatter():
        pltpu.make_async_copy(
            rows_v.at[pl.ds(0, GB)], acc.at[dst_v.at[pl.ds(0, GB)]], sem_sv).wait()
        if with_deg:
            pltpu.make_async_copy(
                ones_v, accd.at[dst_v.at[pl.ds(0, GB)]], sem_sd).wait()

    gather(0, 0)
    for g in range(NGR):
        buf = g % 2
        drain_gather()
        if g + 1 < NGR:
            if g >= 1:
                drain_scatter()
            gather(g + 1, 1 - buf)
        scatter(g, buf)
    drain_scatter()
    drain_scatter()

    @pl.when(wid < NROW - NW * BASE)
    def _():
        pltpu.async_copy(table.at[src_v.at[pl.ds(BASE * CH, CH)]],
                         rows_v.at[pl.ds(0, CH)], sem_g).wait()
        pltpu.sync_copy(rows_v.at[pl.ds(0, CH)],
                        acc.at[dst_v.at[pl.ds(BASE * CH, CH)]], add=True)
        if with_deg:
            pltpu.sync_copy(ones_v.at[pl.ds(0, CH)],
                            accd.at[dst_v.at[pl.ds(BASE * CH, CH)]], add=True)

    plsc.subcore_barrier()
    pltpu.sync_copy(acc.at[pl.ds(sid * RS, RS)], out_acc.at[cid, pl.ds(sid * RS, RS)])
    if with_deg:
        pltpu.sync_copy(accd.at[pl.ds(sid * RS, RS)],
                        out_deg.at[cid, pl.ds(sid * RS, RS)])


@functools.cache
def _make_seg(with_deg):
    mesh = plsc.VectorSubcoreMesh(
        core_axis_name="c", subcore_axis_name="s", num_cores=2, num_subcores=16
    )
    outs = [jax.ShapeDtypeStruct((2, NP, 16), jnp.float32)]
    scratch = [
        pltpu.VMEM(((BASE + 1) * CH,), jnp.int32),
        pltpu.VMEM(((BASE + 1) * CH,), jnp.int32),
        pltpu.VMEM((2 * GB, 16), jnp.float32),
    ]
    if with_deg:
        outs.append(jax.ShapeDtypeStruct((2, NP, 16), jnp.float32))
        scratch.append(pltpu.VMEM((GB, 16), jnp.float32))
    scratch.append(pltpu.VMEM_SHARED((NP, 16), jnp.float32))
    if with_deg:
        scratch.append(pltpu.VMEM_SHARED((NP, 16), jnp.float32))
    else:
        scratch.append(pltpu.VMEM_SHARED((NN, 16), jnp.float32))
    scratch.append(pltpu.SemaphoreType.DMA)
    scratch.append(pltpu.SemaphoreType.DMA)
    if with_deg:
        scratch.append(pltpu.SemaphoreType.DMA)
    return pl.kernel(
        functools.partial(_seg_body, with_deg),
        out_type=tuple(outs) if with_deg else outs[0],
        mesh=mesh,
        scratch_types=scratch,
        compiler_params=pltpu.CompilerParams(use_tc_tiling_on_sc=False),
    )


def _permsc_body(edges3, out, buf):
    cid = lax.axis_index("c")
    sid = lax.axis_index("s")
    wid = sid * 2 + cid
    extra = wid < NROW - NW * BASE

    def row(r, carry):
        for k in range(8):
            v = buf[r, pl.ds(16 * k, 16)]
            q = (v.astype(jnp.float32) * (1.0 / NPK)).astype(jnp.int32)
            buf[r, pl.ds(16 * k, 16)] = (v - q * NPK) * 8 + q
        return carry

    for j in range(2):
        pltpu.sync_copy(edges3.at[pl.ds(wid * BASE, BASE), j], buf.at[pl.ds(0, BASE)])

        @pl.when(extra)
        def _():
            pltpu.sync_copy(edges3.at[pl.ds(NW * BASE + wid, 1), j],
                            buf.at[pl.ds(BASE, 1)])

        lax.fori_loop(0, BASE, row, 0)

        @pl.when(extra)
        def _():
            lax.fori_loop(BASE, BASE + 1, row, 0)

        pltpu.sync_copy(buf.at[pl.ds(0, BASE)], out.at[j, pl.ds(wid * BASE, BASE)])

        @pl.when(extra)
        def _():
            pltpu.sync_copy(buf.at[pl.ds(BASE, 1)],
                            out.at[j, pl.ds(NW * BASE + wid, 1)])


@functools.cache
def _make_perm():
    mesh = plsc.VectorSubcoreMesh(
        core_axis_name="c", subcore_axis_name="s", num_cores=2, num_subcores=16
    )
    return pl.kernel(
        _permsc_body,
        out_type=jax.ShapeDtypeStruct((2, NROW, CH), jnp.int32),
        mesh=mesh,
        scratch_types=[pltpu.VMEM((BASE + 1, CH), jnp.int32)],
        compiler_params=pltpu.CompilerParams(use_tc_tiling_on_sc=False),
    )


def _proj_body(x_ref, wl_ref, wr_ref, xl_ref, xr_ref):
    xv = x_ref[...]
    dn = (((1,), (1,)), ((), ()))
    xls, xrs = [], []
    for a in range(8):
        xa = xv[a]
        xls.append(lax.dot_general(xa, wl_ref[...], dn, preferred_element_type=jnp.float32))
        xrs.append(lax.dot_general(xa, wr_ref[...], dn, preferred_element_type=jnp.float32))
    xl_ref[...] = jnp.concatenate(xls, axis=1)
    xr_ref[...] = jnp.concatenate(xrs, axis=1)


_proj = pl.pallas_call(
    _proj_body,
    out_shape=(
        jax.ShapeDtypeStruct((NPK, 128), jnp.float32),
        jax.ShapeDtypeStruct((NPK, 128), jnp.float32),
    ),
)


def _h_body(a_ref, d_ref, xr_ref, b1_ref, hp_ref, dinv_ref):
    deg = d_ref[0, :NPK] + d_ref[1, :NPK]
    dinv = 1.0 / jnp.maximum(deg, 1.0)
    agg = a_ref[0, :NPK] + a_ref[1, :NPK]
    hp_ref[...] = jnp.maximum(agg * dinv + b1_ref[...] + xr_ref[...], 0.0)
    dinv_ref[...] = dinv


_hcomb = pl.pallas_call(
    _h_body,
    out_shape=(
        jax.ShapeDtypeStruct((NPK, 128), jnp.float32),
        jax.ShapeDtypeStruct((NPK, 128), jnp.float32),
    ),
)


def _out_body(a_ref, dinv_ref, h_ref, w2l_ref, w2r_ref, b2_ref, o_ref):
    m2p = (a_ref[0, :NPK] + a_ref[1, :NPK]) * dinv_ref[...]
    hpv = h_ref[...]
    dn = (((1,), (1,)), ((), ()))
    cols = []
    for a in range(8):
        m2a = m2p[:, 16 * a:16 * (a + 1)]
        ha = hpv[:, 16 * a:16 * (a + 1)]
        z = (lax.dot_general(w2l_ref[...], m2a, dn, preferred_element_type=jnp.float32)
             + lax.dot_general(w2r_ref[...], ha, dn, preferred_element_type=jnp.float32)
             + b2_ref[...])
        m = jnp.max(z, axis=0, keepdims=True)
        lse = jnp.log(jnp.sum(jnp.exp(z - m), axis=0, keepdims=True)) + m
        cols.append(z - lse)
    o_ref[...] = jnp.concatenate(cols, axis=1)


_outk = pl.pallas_call(
    _out_body,
    out_shape=jax.ShapeDtypeStruct((OPc, NN), jnp.float32),
)


def kernel(x, edge_index, W1l, b1, W1r, W2l, b2, W2r):
    edges3 = edge_index.astype(jnp.int32).reshape(2, NROW, CH).swapaxes(0, 1)
    zeros_h = jnp.zeros((NP, 16), jnp.float32)
    ones_h = jnp.ones((GB, 16), jnp.float32)
    edges = _make_perm()(edges3).reshape(2, EE)

    xlp, xrp = _proj(x.reshape(8, NPK, 128), W1l, W1r)
    agg1p, degp = _make_seg(True)(xlp.reshape(NN, 16), edges, zeros_h, ones_h)
    hp, dinvp = _hcomb(
        agg1p.reshape(2, NPP, 128), degp.reshape(2, NPP, 128), xrp,
        jnp.tile(b1, 8).reshape(1, 128))
    agg2p = _make_seg(False)(hp.reshape(NN, 16), edges, zeros_h, ones_h)

    w2l_p = jnp.zeros((OPc, 16), jnp.float32).at[:OO].set(W2l)
    w2r_p = jnp.zeros((OPc, 16), jnp.float32).at[:OO].set(W2r)
    b2c = jnp.full((OPc, 1), -1e30, jnp.float32).at[:OO, 0].set(b2)
    outT = _outk(agg2p.reshape(2, NPP, 128), dinvp, hp, w2l_p, w2r_p, b2c)
    return outT.T[:, :OO]
